# baseline (speedup 1.0000x reference)
import functools

import jax
import jax.numpy as jnp
from jax import lax
from jax.experimental import pallas as pl
from jax.experimental.pallas import tpu as pltpu
from jax.experimental.pallas import tpu_sc as plsc

B = 4096
L = 200
E = 32
N = B * L

NC = 2
NS = 16
NW = NC * NS
PER_W = N // NW
G = 128
S = 512
GPS = S // G
NSTAGES = PER_W // S
NPAIRS = NSTAGES // 2
ROWS_W = PER_W // G

C0, C1, C2, C3, C4 = (0.9999590188675769, -4.932735512906164, 4.041964638154526,
                      -1.2873554659573256, 0.1782067264910494)
S0, S1, S2, S3 = (3.1392768843462933, -5.136388565767432, 2.434666512020243,
                  -0.43779898378705956)

_MESH = plsc.VectorSubcoreMesh(core_axis_name="c", subcore_axis_name="s")


@functools.partial(
    pl.kernel,
    out_type=(jax.ShapeDtypeStruct((E, N), jnp.float32),
              jax.ShapeDtypeStruct((E, N), jnp.float32)),
    mesh=_MESH,
    compiler_params=pltpu.CompilerParams(needs_layout_passes=False,
                                         use_tc_tiling_on_sc=False),
    scratch_types=[
        pltpu.VMEM((ROWS_W, G), jnp.int32),
        pltpu.VMEM((GPS, G), jnp.int32),
        pltpu.VMEM((GPS, G), jnp.int32),
        pltpu.VMEM((S, E), jnp.float32),
        pltpu.VMEM((S, E), jnp.float32),
        pltpu.VMEM((E, S), jnp.float32),
        pltpu.VMEM((E, S), jnp.float32),
        pltpu.VMEM((E, S), jnp.float32),
        pltpu.VMEM((E, S), jnp.float32),
        pltpu.SemaphoreType.DMA,
        pltpu.SemaphoreType.DMA,
        pltpu.SemaphoreType.DMA,
        pltpu.SemaphoreType.DMA,
    ],
)
def _sc_encode(enc_hbm, table_hbm, outr_hbm, outi_hbm,
               enc_v, dec0, dec1, rows0, rows1,
               outr0, outr1, outi0, outi1,
               gsem0, gsem1, osem0, osem1):
    wid = lax.axis_index("s") * NC + lax.axis_index("c")
    base = wid * PER_W
    decs = (dec0, dec1)
    rows = (rows0, rows1)
    outr = (outr0, outr1)
    outi = (outi0, outi1)
    gsems = (gsem0, gsem1)
    osems = (osem0, osem1)

    pltpu.sync_copy(enc_hbm.at[pl.ds(wid * ROWS_W, ROWS_W)], enc_v)

    def decode(s, b):
        for g in range(GPS):
            for v in range(G // 16):
                x = enc_v[s * GPS + g, pl.ds(v * 16, 16)]
                decs[b][g, pl.ds(v * 16, 16)] = lax.shift_right_logical(x, 1)

    def fire_gather(b):
        for g in range(GPS):
            pltpu.async_copy(table_hbm.at[decs[b].at[g]],
                             rows[b].at[pl.ds(g * G, G)], gsems[b])

    def drain_gather(b):
        pltpu.make_async_copy(table_hbm.at[pl.ds(0, S)], rows[b], gsems[b]).wait()

    def fire_out(s, b):
        for e in range(E):
            pltpu.async_copy(outr[b].at[e],
                             outr_hbm.at[e, pl.ds(base + s * S, S)], osems[b])
            pltpu.async_copy(outi[b].at[e],
                             outi_hbm.at[e, pl.ds(base + s * S, S)], osems[b])

    def drain_out(b):
        pltpu.make_async_copy(outr[b], outr_hbm.at[pl.ds(0, E), pl.ds(0, S)],
                              osems[b]).wait()
        pltpu.make_async_copy(outi[b], outi_hbm.at[pl.ds(0, E), pl.ds(0, S)],
                              osems[b]).wait()

    def compute_stage(s, b):
        iota = lax.iota(jnp.int32, 16)

        def chunk_body(pc, carry):
            enc = enc_v[s * GPS + lax.shift_right_logical(pc, 3),
                        pl.ds((pc & 7) * 16, 16)]
            amp = 1.0 - (enc & 1).astype(jnp.float32)
            p_idx = iota + pc * 16

            def e_body(e, carry2):
                e_idx = jnp.full((16,), e, dtype=jnp.int32)
                x = plsc.load_gather(rows[b], [p_idx, e_idx])
                ex = jnp.exp(x + x)
                t = 1.0 - 2.0 / (ex + 1.0)
                z = t * t
                cv = C0 + z * (C1 + z * (C2 + z * (C3 + z * C4)))
                sv = t * (S0 + z * (S1 + z * (S2 + z * S3)))
                outr[b][e, pl.ds(pc * 16, 16)] = cv * amp
                outi[b][e, pl.ds(pc * 16, 16)] = sv * amp
                return carry2
            lax.fori_loop(0, E, e_body, 0, unroll=2)
            return carry
        lax.fori_loop(0, S // 16, chunk_body, 0)

    decode(0, 0)
    fire_gather(0)
    decode(1, 1)
    fire_gather(1)

    for b in (0, 1):
        drain_gather(b)
        compute_stage(b, b)
        fire_out(b, b)
        decode(b + 2, b)
        fire_gather(b)

    def pair_body(p, carry):
        for b in (0, 1):
            s = 2 * p + b
            drain_gather(b)
            drain_out(b)
            compute_stage(s, b)
            fire_out(s, b)
            decode(s + 2, b)
            fire_gather(b)
        return carry
    lax.fori_loop(1, NPAIRS - 1, pair_body, 0)

    for b in (0, 1):
        s = 2 * (NPAIRS - 1) + b
        drain_gather(b)
        drain_out(b)
        compute_stage(s, b)
        fire_out(s, b)

    drain_out(0)
    drain_out(1)


def kernel(input_ids, mask, table):
    enc = (input_ids.astype(jnp.int32) * 2 + mask.astype(jnp.int32))
    enc2d = enc.reshape(N // G, G)
    out_r, out_i = _sc_encode(enc2d, table)
    re = out_r.T.reshape(B, L, E)
    im = out_i.T.reshape(B, L, E)
    return lax.complex(re, im)


# speedup vs baseline: 1.9212x; 1.9212x over previous
import functools

import jax
import jax.numpy as jnp
from jax import lax
from jax.experimental import pallas as pl
from jax.experimental.pallas import tpu as pltpu
from jax.experimental.pallas import tpu_sc as plsc
import numpy as np

B = 4096
L = 200
E = 32
N = B * L
NCHUNK = 2
NK = N // NCHUNK
BK = B // NCHUNK

NC = 2
NS = 16
NW = NC * NS
PER_W = NK // NW
G = 128
S = 256
GPS = S // G
NSTAGES = PER_W // S
NPAIRS = NSTAGES // 2
ROWS_W = PER_W // G

C0, C1, C2, C3, C4 = (0.9999590188675769, -4.932735512906164, 4.041964638154526,
                      -1.2873554659573256, 0.1782067264910494)
S0, S1, S2, S3 = (3.1392768843462933, -5.136388565767432, 2.434666512020243,
                  -0.43779898378705956)

_MESH = plsc.VectorSubcoreMesh(core_axis_name="c", subcore_axis_name="s")


@functools.partial(
    pl.kernel,
    out_type=(jax.ShapeDtypeStruct((4, NK // G, 8, G), jnp.float32),
              jax.ShapeDtypeStruct((4, NK // G, 8, G), jnp.float32)),
    mesh=_MESH,
    compiler_params=pltpu.CompilerParams(needs_layout_passes=False,
                                         use_tc_tiling_on_sc=False),
    scratch_types=[
        pltpu.VMEM((ROWS_W, G + 16), jnp.int32),
        pltpu.VMEM((GPS, G), jnp.int32),
        pltpu.VMEM((GPS, G), jnp.int32),
        pltpu.VMEM((S, E), jnp.float32),
        pltpu.VMEM((S, E), jnp.float32),
        pltpu.VMEM((4, GPS, 8, G), jnp.float32),
        pltpu.VMEM((4, GPS, 8, G), jnp.float32),
        pltpu.VMEM((4, GPS, 8, G), jnp.float32),
        pltpu.VMEM((4, GPS, 8, G), jnp.float32),
        pltpu.SemaphoreType.DMA,
        pltpu.SemaphoreType.DMA,
        pltpu.SemaphoreType.DMA,
        pltpu.SemaphoreType.DMA,
    ],
)
def _sc_encode(enc_hbm, table_hbm, outr_hbm, outi_hbm,
               enc_v, dec0, dec1, rows0, rows1,
               outr0, outr1, outi0, outi1,
               gsem0, gsem1, osem0, osem1):
    wid = lax.axis_index("s") * NC + lax.axis_index("c")
    decs = (dec0, dec1)
    rows = (rows0, rows1)
    outr = (outr0, outr1)
    outi = (outi0, outi1)
    gsems = (gsem0, gsem1)
    osems = (osem0, osem1)

    pltpu.sync_copy(enc_hbm.at[pl.ds(wid * ROWS_W, ROWS_W)],
                    enc_v.at[pl.ds(0, ROWS_W), pl.ds(0, G)])

    def decode(s, b):
        for g in range(GPS):
            for v in range(G // 16):
                x = enc_v[s * GPS + g, pl.ds(v * 16, 16)]
                decs[b][g, pl.ds(v * 16, 16)] = lax.shift_right_logical(x, 1)

    def fire_gather(b):
        for g in range(GPS):
            pltpu.async_copy(table_hbm.at[decs[b].at[g]],
                             rows[b].at[pl.ds(g * G, G)], gsems[b])

    def drain_gather(b):
        pltpu.make_async_copy(table_hbm.at[pl.ds(0, S)], rows[b], gsems[b]).wait()

    def fire_out(s, b):
        ct0 = wid * ROWS_W + s * GPS
        for rt in range(4):
            pltpu.async_copy(outr[b].at[rt],
                             outr_hbm.at[rt, pl.ds(ct0, GPS)], osems[b])
            pltpu.async_copy(outi[b].at[rt],
                             outi_hbm.at[rt, pl.ds(ct0, GPS)], osems[b])

    def drain_out(b):
        pltpu.make_async_copy(outr[b], outr_hbm.at[pl.ds(0, 4), pl.ds(0, GPS)],
                              osems[b]).wait()
        pltpu.make_async_copy(outi[b], outi_hbm.at[pl.ds(0, 4), pl.ds(0, GPS)],
                              osems[b]).wait()

    def compute_stage(s, b):
        iota = lax.iota(jnp.int32, 16)
        rt0 = lax.shift_right_logical(iota, 3)
        e0 = iota & 7
        rt1 = lax.shift_right_logical(iota + 16, 3)
        e1 = (iota + 16) & 7

        def row_body(p, carry):
            enc = enc_v[s * GPS + lax.shift_right_logical(p, 7), pl.ds(p & 127, 16)]
            amp = jnp.full((16,), 1.0 - (enc[0] & 1).astype(jnp.float32),
                           dtype=jnp.float32)
            ct = jnp.full((16,), lax.shift_right_logical(p, 7), dtype=jnp.int32)
            col = jnp.full((16,), p & 127, dtype=jnp.int32)
            for half, (rtv, ev) in ((0, (rt0, e0)), (1, (rt1, e1))):
                x = rows[b][p, pl.ds(16 * half, 16)]
                ex = jnp.exp(x + x)
                t = 1.0 - 2.0 / (ex + 1.0)
                z = t * t
                cv = C0 + z * (C1 + z * (C2 + z * (C3 + z * C4)))
                sv = t * (S0 + z * (S1 + z * (S2 + z * S3)))
                plsc.store_scatter(outr[b], [rtv, ct, ev, col], cv * amp)
                plsc.store_scatter(outi[b], [rtv, ct, ev, col], sv * amp)
            return carry
        lax.fori_loop(0, S, row_body, 0)

    decode(0, 0)
    fire_gather(0)
    decode(1, 1)
    fire_gather(1)

    for b in (0, 1):
        drain_gather(b)
        compute_stage(b, b)
        fire_out(b, b)
        decode(b + 2, b)
        fire_gather(b)

    def pair_body(p, carry):
        for b in (0, 1):
            s = 2 * p + b
            drain_gather(b)
            drain_out(b)
            compute_stage(s, b)
            fire_out(s, b)
            decode(s + 2, b)
            fire_gather(b)
        return carry
    lax.fori_loop(1, NPAIRS - 1, pair_body, 0)

    for b in (0, 1):
        s = 2 * (NPAIRS - 1) + b
        drain_gather(b)
        drain_out(b)
        compute_stage(s, b)
        fire_out(s, b)

    drain_out(0)
    drain_out(1)


def kernel(input_ids, mask, table):
    enc = (input_ids.astype(jnp.int32) * 2 + mask.astype(jnp.int32))
    enc2d = enc.reshape(N // G, G)
    chunks = []
    for k in range(NCHUNK):
        enc_k = lax.slice(enc2d, (k * (NK // G), 0), ((k + 1) * (NK // G), G))
        r4, i4 = _sc_encode(enc_k, table)
        re = r4.transpose(0, 2, 1, 3).reshape(E, NK).T.reshape(BK, L, E)
        im = i4.transpose(0, 2, 1, 3).reshape(E, NK).T.reshape(BK, L, E)
        chunks.append(lax.complex(re, im))
    return jnp.concatenate(chunks, axis=0)


# speedup vs baseline: 2.2030x; 1.1467x over previous
import functools

import jax
import jax.numpy as jnp
from jax import lax
from jax.experimental import pallas as pl
from jax.experimental.pallas import tpu as pltpu
from jax.experimental.pallas import tpu_sc as plsc
import numpy as np

B = 4096
L = 200
E = 32
N = B * L
NCHUNK = 2
NK = N // NCHUNK
BK = B // NCHUNK

NC = 2
NS = 16
NW = NC * NS
PER_W = NK // NW
G = 128
S = 256
GPS = S // G
NSTAGES = PER_W // S
NPAIRS = NSTAGES // 2
ROWS_W = PER_W // G

C0, C1, C2, C3, C4 = (0.9999590188675769, -4.932735512906164, 4.041964638154526,
                      -1.2873554659573256, 0.1782067264910494)
S0, S1, S2, S3 = (3.1392768843462933, -5.136388565767432, 2.434666512020243,
                  -0.43779898378705956)

_MESH = plsc.VectorSubcoreMesh(core_axis_name="c", subcore_axis_name="s")


@functools.partial(
    pl.kernel,
    out_type=(jax.ShapeDtypeStruct((4, NK // G, 8, G), jnp.float32),
              jax.ShapeDtypeStruct((4, NK // G, 8, G), jnp.float32)),
    mesh=_MESH,
    compiler_params=pltpu.CompilerParams(needs_layout_passes=False,
                                         use_tc_tiling_on_sc=False),
    scratch_types=[
        pltpu.VMEM((ROWS_W, G + 16), jnp.int32),
        pltpu.VMEM((GPS, G), jnp.int32),
        pltpu.VMEM((GPS, G), jnp.int32),
        pltpu.VMEM((S, E), jnp.float32),
        pltpu.VMEM((S, E), jnp.float32),
        pltpu.VMEM((4, GPS, 8, G + 1), jnp.float32),
        pltpu.VMEM((4, GPS, 8, G + 1), jnp.float32),
        pltpu.VMEM((4, GPS, 8, G + 1), jnp.float32),
        pltpu.VMEM((4, GPS, 8, G + 1), jnp.float32),
        pltpu.SemaphoreType.DMA,
        pltpu.SemaphoreType.DMA,
        pltpu.SemaphoreType.DMA,
        pltpu.SemaphoreType.DMA,
    ],
)
def _sc_encode(enc_hbm, table_hbm, outr_hbm, outi_hbm,
               enc_v, dec0, dec1, rows0, rows1,
               outr0, outr1, outi0, outi1,
               gsem0, gsem1, osem0, osem1):
    wid = lax.axis_index("s") * NC + lax.axis_index("c")
    decs = (dec0, dec1)
    rows = (rows0, rows1)
    outr = (outr0, outr1)
    outi = (outi0, outi1)
    gsems = (gsem0, gsem1)
    osems = (osem0, osem1)

    pltpu.sync_copy(enc_hbm.at[pl.ds(wid * ROWS_W, ROWS_W)],
                    enc_v.at[pl.ds(0, ROWS_W), pl.ds(0, G)])

    def decode(s, b):
        for g in range(GPS):
            for v in range(G // 16):
                x = enc_v[s * GPS + g, pl.ds(v * 16, 16)]
                decs[b][g, pl.ds(v * 16, 16)] = lax.shift_right_logical(x, 1)

    def fire_gather(b):
        for g in range(GPS):
            pltpu.async_copy(table_hbm.at[decs[b].at[g]],
                             rows[b].at[pl.ds(g * G, G)], gsems[b])

    def drain_gather(b):
        pltpu.make_async_copy(table_hbm.at[pl.ds(0, S)], rows[b], gsems[b]).wait()

    def fire_out(s, b):
        ct0 = wid * ROWS_W + s * GPS
        for rt in range(4):
            pltpu.async_copy(outr[b].at[rt, pl.ds(0, GPS), pl.ds(0, 8), pl.ds(0, G)],
                             outr_hbm.at[rt, pl.ds(ct0, GPS)], osems[b])
            pltpu.async_copy(outi[b].at[rt, pl.ds(0, GPS), pl.ds(0, 8), pl.ds(0, G)],
                             outi_hbm.at[rt, pl.ds(ct0, GPS)], osems[b])

    def drain_out(b):
        pltpu.make_async_copy(outr[b].at[pl.ds(0, 4), pl.ds(0, GPS), pl.ds(0, 8),
                                         pl.ds(0, G)],
                              outr_hbm.at[pl.ds(0, 4), pl.ds(0, GPS)],
                              osems[b]).wait()
        pltpu.make_async_copy(outi[b].at[pl.ds(0, 4), pl.ds(0, GPS), pl.ds(0, 8),
                                         pl.ds(0, G)],
                              outi_hbm.at[pl.ds(0, 4), pl.ds(0, GPS)],
                              osems[b]).wait()

    def compute_stage(s, b):
        iota = lax.iota(jnp.int32, 16)
        rt0 = lax.shift_right_logical(iota, 3)
        e0 = iota & 7
        rt1 = lax.shift_right_logical(iota + 16, 3)
        e1 = (iota + 16) & 7

        def row_body(p, carry):
            enc = enc_v[s * GPS + lax.shift_right_logical(p, 7), pl.ds(p & 127, 16)]
            amp = jnp.full((16,), 1.0 - (enc[0] & 1).astype(jnp.float32),
                           dtype=jnp.float32)
            ct = jnp.full((16,), lax.shift_right_logical(p, 7), dtype=jnp.int32)
            col = jnp.full((16,), p & 127, dtype=jnp.int32)
            for half, (rtv, ev) in ((0, (rt0, e0)), (1, (rt1, e1))):
                x = rows[b][p, pl.ds(16 * half, 16)]
                ex = jnp.exp(x + x)
                t = 1.0 - 2.0 / (ex + 1.0)
                z = t * t
                cv = C0 + z * (C1 + z * (C2 + z * (C3 + z * C4)))
                sv = t * (S0 + z * (S1 + z * (S2 + z * S3)))
                plsc.store_scatter(outr[b], [rtv, ct, ev, col], cv * amp)
                plsc.store_scatter(outi[b], [rtv, ct, ev, col], sv * amp)
            return carry
        lax.fori_loop(0, S, row_body, 0)

    decode(0, 0)
    fire_gather(0)
    decode(1, 1)
    fire_gather(1)

    for b in (0, 1):
        drain_gather(b)
        compute_stage(b, b)
        fire_out(b, b)
        decode(b + 2, b)
        fire_gather(b)

    def pair_body(p, carry):
        for b in (0, 1):
            s = 2 * p + b
            drain_gather(b)
            drain_out(b)
            compute_stage(s, b)
            fire_out(s, b)
            decode(s + 2, b)
            fire_gather(b)
        return carry
    lax.fori_loop(1, NPAIRS - 1, pair_body, 0)

    for b in (0, 1):
        s = 2 * (NPAIRS - 1) + b
        drain_gather(b)
        drain_out(b)
        compute_stage(s, b)
        fire_out(s, b)

    drain_out(0)
    drain_out(1)


def kernel(input_ids, mask, table):
    enc = (input_ids.astype(jnp.int32) * 2 + mask.astype(jnp.int32))
    enc2d = enc.reshape(N // G, G)
    chunks = []
    for k in range(NCHUNK):
        enc_k = lax.slice(enc2d, (k * (NK // G), 0), ((k + 1) * (NK // G), G))
        r4, i4 = _sc_encode(enc_k, table)
        re = r4.transpose(0, 2, 1, 3).reshape(E, NK).T.reshape(BK, L, E)
        im = i4.transpose(0, 2, 1, 3).reshape(E, NK).T.reshape(BK, L, E)
        chunks.append(lax.complex(re, im))
    return jnp.concatenate(chunks, axis=0)
